# Initial kernel scaffold; baseline (speedup 1.0000x reference)
import jax
import jax.numpy as jnp
from jax.experimental import pallas as pl


def kernel(x, W1f, W1d, W2f, W2d, W3f, W3d, W4f, W4d, W5):
    raise NotImplementedError("write your pallas kernel here")


# speedup vs baseline: 2.0606x; 2.0606x over previous
import functools

import jax
import jax.numpy as jnp
from jax.experimental import pallas as pl
from jax.experimental.pallas import tpu as pltpu

EPS = 1e-6
B, N, K = 4, 2048, 20
CH = 128


def _dot(a, b):
    return jax.lax.dot_general(a, b, (((1,), (0,)), ((), ())),
                               preferred_element_type=jnp.float32)


def _l1_feat(xt_ref, nbr_ref, sl):
    nx = nbr_ref[:, 0, :, sl]
    ny = nbr_ref[:, 1, :, sl]
    nz = nbr_ref[:, 2, :, sl]
    cx = xt_ref[:, 0, sl][:, None, :]
    cy = xt_ref[:, 1, sl][:, None, :]
    cz = xt_ref[:, 2, sl][:, None, :]
    ex, ey, ez = nx - cx, ny - cy, nz - cz
    rx = ny * cz - nz * cy
    ry = nz * cx - nx * cz
    rz = nx * cy - ny * cx
    bc = jnp.broadcast_to
    shp = ex.shape
    F = jnp.stack([
        jnp.stack([ex, ey, ez]),
        jnp.stack([bc(cx, shp), bc(cy, shp), bc(cz, shp)]),
        jnp.stack([rx, ry, rz]),
    ])
    return F.reshape(3, -1)


def _k3a_body(xt_ref, nbr_ref, w1f_ref, out_ref):
    def chunk(t, acc):
        sl = pl.ds(t * CH, CH)
        Ff = _l1_feat(xt_ref, nbr_ref, sl)
        p = _dot(w1f_ref[...], Ff).reshape(64, 3, B, K, CH)
        nrm = jnp.sqrt(p[:, 0] ** 2 + p[:, 1] ** 2 + p[:, 2] ** 2) + EPS
        return (acc[0] + jnp.sum(nrm, axis=(1, 2, 3)),
                acc[1] + jnp.sum(nrm * nrm, axis=(1, 2, 3)))

    z = jnp.zeros((64,), jnp.float32)
    s1, s2 = jax.lax.fori_loop(0, N // CH, chunk, (z, z))
    out_ref[...] = jnp.stack([s1, s2])


def _k3b_body(xt_ref, nbr_ref, w1f_ref, w1d_ref, st_ref, out_ref):
    cnt = float(B * N * K)
    mu = st_ref[0, :64] / cnt
    var = st_ref[1, :64] / cnt - mu * mu
    inv_sig = jax.lax.rsqrt(var + 1e-5)

    def chunk(t, _):
        sl = pl.ds(t * CH, CH)
        Ff = _l1_feat(xt_ref, nbr_ref, sl)
        p = _dot(w1f_ref[...], Ff).reshape(64, 3, B, K, CH)
        d = _dot(w1d_ref[...], Ff).reshape(64, 3, B, K, CH)
        nrm = jnp.sqrt(p[:, 0] ** 2 + p[:, 1] ** 2 + p[:, 2] ** 2) + EPS
        s = (nrm - mu[:, None, None, None]) * inv_sig[:, None, None, None] / nrm
        ps = p * s[:, None]
        dot0 = jnp.sum(ps * d, axis=1)
        dsq = d[:, 0] ** 2 + d[:, 1] ** 2 + d[:, 2] ** 2
        coef = jnp.where(dot0 < 0, dot0 / (dsq + EPS), 0.0)
        o = ps - coef[:, None] * d
        h = jnp.mean(o, axis=3)
        out_ref[:, :, :, sl] = jnp.transpose(h, (0, 2, 1, 3))
        return 0

    jax.lax.fori_loop(0, N // CH, chunk, 0)


TC4 = 512
M4 = B * 3 * N


def _lbr_ref(h_ref, p_ref, d_ref, cin, cout, wf, wd):
    def mm(t, _):
        sl = pl.ds(t * TC4, TC4)
        hc = h_ref[:cin, sl]
        p_ref[:cout, sl] = _dot(wf, hc)
        d_ref[:cout, sl] = _dot(wd, hc)
        return 0

    jax.lax.fori_loop(0, M4 // TC4, mm, 0)

    def stat(bt, acc):
        b = bt // (N // TC4)
        t = bt % (N // TC4)
        base = b * 3 * N + t * TC4
        px = p_ref[:cout, pl.ds(base, TC4)]
        py = p_ref[:cout, pl.ds(base + N, TC4)]
        pz = p_ref[:cout, pl.ds(base + 2 * N, TC4)]
        nrm = jnp.sqrt(px * px + py * py + pz * pz) + EPS
        return (acc[0] + jnp.sum(nrm, axis=1),
                acc[1] + jnp.sum(nrm * nrm, axis=1))

    z = jnp.zeros((cout,), jnp.float32)
    sn, sn2 = jax.lax.fori_loop(0, B * (N // TC4), stat, (z, z))
    cnt = float(B * N)
    mu = sn / cnt
    var = sn2 / cnt - mu * mu
    inv_sig = jax.lax.rsqrt(var + 1e-5)

    def apply(bt, _):
        b = bt // (N // TC4)
        t = bt % (N // TC4)
        base = b * 3 * N + t * TC4
        sx, sy, sz = (pl.ds(base, TC4), pl.ds(base + N, TC4),
                      pl.ds(base + 2 * N, TC4))
        px, py, pz = p_ref[:cout, sx], p_ref[:cout, sy], p_ref[:cout, sz]
        dx, dy, dz = d_ref[:cout, sx], d_ref[:cout, sy], d_ref[:cout, sz]
        nrm = jnp.sqrt(px * px + py * py + pz * pz) + EPS
        s = (nrm - mu[:, None]) * inv_sig[:, None] / nrm
        px, py, pz = px * s, py * s, pz * s
        dot0 = px * dx + py * dy + pz * dz
        dsq = dx * dx + dy * dy + dz * dz
        coef = jnp.where(dot0 < 0, dot0 / (dsq + EPS), 0.0)
        h_ref[:cout, sx] = px - coef * dx
        h_ref[:cout, sy] = py - coef * dy
        h_ref[:cout, sz] = pz - coef * dz
        return 0

    jax.lax.fori_loop(0, B * (N // TC4), apply, 0)


def _k4_body(h1_ref, w2f_ref, w2d_ref, w3f_ref, w3d_ref, w4f_ref, w4d_ref,
             w5_ref, out_ref, h_ref, p_ref, d_ref):
    h_ref[:64, :] = h1_ref[...]
    _lbr_ref(h_ref, p_ref, d_ref, 64, 64, w2f_ref[...], w2d_ref[...])
    _lbr_ref(h_ref, p_ref, d_ref, 64, 64, w3f_ref[...], w3d_ref[...])
    _lbr_ref(h_ref, p_ref, d_ref, 64, 128, w4f_ref[...], w4d_ref[...])

    T5 = 512
    w5 = w5_ref[...]

    def chunk(bt, acc):
        s1, s2, sn, sn2 = acc
        b = bt // (N // T5)
        t = bt % (N // T5)
        base = b * 3 * N + t * T5
        hx = h_ref[:, pl.ds(base, T5)]
        hy = h_ref[:, pl.ds(base + N, T5)]
        hz = h_ref[:, pl.ds(base + 2 * N, T5)]
        h5x, h5y, h5z = _dot(w5, hx), _dot(w5, hy), _dot(w5, hz)
        nrm = jnp.sqrt(h5x * h5x + h5y * h5y + h5z * h5z) + EPS
        rin = 1.0 / nrm
        upd1 = jnp.stack([jnp.sum(h5x, axis=1), jnp.sum(h5y, axis=1),
                          jnp.sum(h5z, axis=1)], axis=1)
        upd2 = jnp.stack([jnp.sum(h5x * rin, axis=1), jnp.sum(h5y * rin, axis=1),
                          jnp.sum(h5z * rin, axis=1)], axis=1)
        bsel = (jax.lax.broadcasted_iota(jnp.int32, (1024, B * 3), 1) // 3
                == b).astype(jnp.float32)
        pad = jnp.zeros((1024, B * 3), jnp.float32)
        upd1 = pad + jnp.tile(upd1, (1, B)) * bsel
        upd2 = pad + jnp.tile(upd2, (1, B)) * bsel
        return (s1 + upd1, s2 + upd2,
                sn + jnp.sum(nrm, axis=1), sn2 + jnp.sum(nrm * nrm, axis=1))

    z13 = jnp.zeros((1024, B * 3), jnp.float32)
    z1 = jnp.zeros((1024,), jnp.float32)
    s1, s2, sn, sn2 = jax.lax.fori_loop(0, B * (N // T5), chunk,
                                        (z13, z13, z1, z1))
    cnt = float(B * N)
    mu = sn / cnt
    var = sn2 / cnt - mu * mu
    scale = jax.lax.rsqrt(var + 1e-5) / float(N)
    res = (s1 - mu[:, None] * s2) * scale[:, None]
    out_ref[...] = jnp.transpose(res.reshape(1024, B, 3), (1, 0, 2))


TK1 = 256
NSURV = 6


def _k1_body(xt_ref, xtile_ref, out_ref):
    IMIN = jnp.int32(-2147483648)
    xb = xt_ref[0]
    xtile = xtile_ref[0]
    pdt = _dot(jnp.transpose(xtile), xb) * 2.0
    xx_all = jnp.sum(xb * xb, axis=0, keepdims=True)
    xx_t = jnp.sum(xtile * xtile, axis=0, keepdims=True)
    pd = pdt - jnp.transpose(xx_t) - xx_all
    bits = jax.lax.bitcast_convert_type(pd, jnp.int32)
    key = jnp.where(bits >= 0, bits, bits ^ jnp.int32(0x7FFFFFFF))
    chunks = [key[:, c * 128:(c + 1) * 128] for c in range(N // 128)]
    lane = jax.lax.broadcasted_iota(jnp.int32, (TK1, 128), 1)
    ridx = [(N - 1) - c * 128 - lane for c in range(N // 128)]

    svs, svi = [], []
    for s in range(NSURV):
        m = functools.reduce(jnp.maximum, chunks)
        mi = functools.reduce(
            jnp.maximum,
            [jnp.where(v == m, r, IMIN) for v, r in zip(chunks, ridx)])
        svs.append(m)
        svi.append(mi)
        if s < NSURV - 1:
            chunks = [jnp.where(r == mi, IMIN, v)
                      for v, r in zip(chunks, ridx)]

    idxs = []
    for r in range(K):
        m6 = functools.reduce(jnp.maximum, svs)
        w = jnp.max(m6, axis=1, keepdims=True)
        wi = jnp.max(
            functools.reduce(
                jnp.maximum,
                [jnp.where(v == w, i, IMIN) for v, i in zip(svs, svi)]),
            axis=1, keepdims=True)
        idxs.append((N - 1) - wi)
        if r < K - 1:
            svs = [jnp.where(i == wi, IMIN, v) for v, i in zip(svs, svi)]
    out_ref[0] = jnp.concatenate(idxs, axis=1)


def _knn_idx(xt):
    return pl.pallas_call(
        _k1_body,
        grid=(B, N // TK1),
        in_specs=[
            pl.BlockSpec((1, 3, N), lambda b, j: (b, 0, 0)),
            pl.BlockSpec((1, 3, TK1), lambda b, j: (b, 0, j)),
        ],
        out_specs=pl.BlockSpec((1, TK1, K), lambda b, j: (b, j, 0)),
        out_shape=jax.ShapeDtypeStruct((B, N, K), jnp.int32),
    )(xt, xt)


def kernel(x, W1f, W1d, W2f, W2d, W3f, W3d, W4f, W4d, W5):
    f32 = jnp.float32
    xt = jnp.transpose(x, (0, 2, 1))

    idx = _knn_idx(xt)

    nbr = x[jnp.arange(B)[:, None, None], idx]
    nbrT = jnp.transpose(nbr, (0, 3, 2, 1))

    stats = pl.pallas_call(
        _k3a_body,
        out_shape=jax.ShapeDtypeStruct((2, 64), f32),
    )(xt, nbrT, W1f)
    h1 = pl.pallas_call(
        _k3b_body,
        out_shape=jax.ShapeDtypeStruct((64, B, 3, N), f32),
    )(xt, nbrT, W1f, W1d, stats)

    out = pl.pallas_call(
        _k4_body,
        out_shape=jax.ShapeDtypeStruct((B, 1024, 3), f32),
        scratch_shapes=[
            pltpu.VMEM((128, M4), f32),
            pltpu.VMEM((128, M4), f32),
            pltpu.VMEM((128, M4), f32),
        ],
    )(h1.reshape(64, M4), W2f, W2d, W3f, W3d, W4f, W4d, W5)
    return out


# speedup vs baseline: 7.5298x; 3.6542x over previous
import functools

import jax
import jax.numpy as jnp
from jax import lax
from jax.experimental import pallas as pl
from jax.experimental.pallas import tpu as pltpu
from jax.experimental.pallas import tpu_sc as plsc

EPS = 1e-6
B, N, K = 4, 2048, 20
CH = 128


def _dot(a, b):
    return jax.lax.dot_general(a, b, (((1,), (0,)), ((), ())),
                               preferred_element_type=jnp.float32)


def _l1_feat(xt_ref, nbr_ref, sl):
    nx = nbr_ref[0, :, :, sl]
    ny = nbr_ref[1, :, :, sl]
    nz = nbr_ref[2, :, :, sl]
    cx = xt_ref[:, 0, sl][:, None, :]
    cy = xt_ref[:, 1, sl][:, None, :]
    cz = xt_ref[:, 2, sl][:, None, :]
    ex, ey, ez = nx - cx, ny - cy, nz - cz
    rx = ny * cz - nz * cy
    ry = nz * cx - nx * cz
    rz = nx * cy - ny * cx
    bc = jnp.broadcast_to
    shp = ex.shape
    F = jnp.stack([
        jnp.stack([ex, ey, ez]),
        jnp.stack([bc(cx, shp), bc(cy, shp), bc(cz, shp)]),
        jnp.stack([rx, ry, rz]),
    ])
    return F.reshape(3, -1)


def _k3a_body(xt_ref, nbr_ref, w1f_ref, out_ref):
    def chunk(t, acc):
        sl = pl.ds(t * CH, CH)
        Ff = _l1_feat(xt_ref, nbr_ref, sl)
        p = _dot(w1f_ref[...], Ff).reshape(64, 3, B, K, CH)
        nrm = jnp.sqrt(p[:, 0] ** 2 + p[:, 1] ** 2 + p[:, 2] ** 2) + EPS
        return (acc[0] + jnp.sum(nrm, axis=(1, 2, 3)),
                acc[1] + jnp.sum(nrm * nrm, axis=(1, 2, 3)))

    z = jnp.zeros((64,), jnp.float32)
    s1, s2 = jax.lax.fori_loop(0, N // CH, chunk, (z, z))
    out_ref[...] = jnp.stack([s1, s2])


def _k3b_body(xt_ref, nbr_ref, w1f_ref, w1d_ref, st_ref, out_ref):
    cnt = float(B * N * K)
    mu = st_ref[0, :64] / cnt
    var = st_ref[1, :64] / cnt - mu * mu
    inv_sig = jax.lax.rsqrt(var + 1e-5)

    def chunk(t, _):
        sl = pl.ds(t * CH, CH)
        Ff = _l1_feat(xt_ref, nbr_ref, sl)
        p = _dot(w1f_ref[...], Ff).reshape(64, 3, B, K, CH)
        d = _dot(w1d_ref[...], Ff).reshape(64, 3, B, K, CH)
        nrm = jnp.sqrt(p[:, 0] ** 2 + p[:, 1] ** 2 + p[:, 2] ** 2) + EPS
        s = (nrm - mu[:, None, None, None]) * inv_sig[:, None, None, None] / nrm
        ps = p * s[:, None]
        dot0 = jnp.sum(ps * d, axis=1)
        dsq = d[:, 0] ** 2 + d[:, 1] ** 2 + d[:, 2] ** 2
        coef = jnp.where(dot0 < 0, dot0 / (dsq + EPS), 0.0)
        o = ps - coef[:, None] * d
        h = jnp.mean(o, axis=3)
        out_ref[:, :, :, sl] = jnp.transpose(h, (0, 2, 1, 3))
        return 0

    jax.lax.fori_loop(0, N // CH, chunk, 0)


TC4 = 512
M4 = B * 3 * N


def _lbr_ref(h_ref, p_ref, d_ref, cin, cout, wf, wd):
    def mm(t, _):
        sl = pl.ds(t * TC4, TC4)
        hc = h_ref[:cin, sl]
        p_ref[:cout, sl] = _dot(wf, hc)
        d_ref[:cout, sl] = _dot(wd, hc)
        return 0

    jax.lax.fori_loop(0, M4 // TC4, mm, 0)

    def stat(bt, acc):
        b = bt // (N // TC4)
        t = bt % (N // TC4)
        base = b * 3 * N + t * TC4
        px = p_ref[:cout, pl.ds(base, TC4)]
        py = p_ref[:cout, pl.ds(base + N, TC4)]
        pz = p_ref[:cout, pl.ds(base + 2 * N, TC4)]
        nrm = jnp.sqrt(px * px + py * py + pz * pz) + EPS
        return (acc[0] + jnp.sum(nrm, axis=1),
                acc[1] + jnp.sum(nrm * nrm, axis=1))

    z = jnp.zeros((cout,), jnp.float32)
    sn, sn2 = jax.lax.fori_loop(0, B * (N // TC4), stat, (z, z))
    cnt = float(B * N)
    mu = sn / cnt
    var = sn2 / cnt - mu * mu
    inv_sig = jax.lax.rsqrt(var + 1e-5)

    def apply(bt, _):
        b = bt // (N // TC4)
        t = bt % (N // TC4)
        base = b * 3 * N + t * TC4
        sx, sy, sz = (pl.ds(base, TC4), pl.ds(base + N, TC4),
                      pl.ds(base + 2 * N, TC4))
        px, py, pz = p_ref[:cout, sx], p_ref[:cout, sy], p_ref[:cout, sz]
        dx, dy, dz = d_ref[:cout, sx], d_ref[:cout, sy], d_ref[:cout, sz]
        nrm = jnp.sqrt(px * px + py * py + pz * pz) + EPS
        s = (nrm - mu[:, None]) * inv_sig[:, None] / nrm
        px, py, pz = px * s, py * s, pz * s
        dot0 = px * dx + py * dy + pz * dz
        dsq = dx * dx + dy * dy + dz * dz
        coef = jnp.where(dot0 < 0, dot0 / (dsq + EPS), 0.0)
        h_ref[:cout, sx] = px - coef * dx
        h_ref[:cout, sy] = py - coef * dy
        h_ref[:cout, sz] = pz - coef * dz
        return 0

    jax.lax.fori_loop(0, B * (N // TC4), apply, 0)


def _k4_body(h1_ref, w2f_ref, w2d_ref, w3f_ref, w3d_ref, w4f_ref, w4d_ref,
             w5_ref, out_ref, h_ref, p_ref, d_ref):
    h_ref[:64, :] = h1_ref[...]
    _lbr_ref(h_ref, p_ref, d_ref, 64, 64, w2f_ref[...], w2d_ref[...])
    _lbr_ref(h_ref, p_ref, d_ref, 64, 64, w3f_ref[...], w3d_ref[...])
    _lbr_ref(h_ref, p_ref, d_ref, 64, 128, w4f_ref[...], w4d_ref[...])

    T5 = 512
    w5 = w5_ref[...]

    def chunk(bt, acc):
        s1, s2, sn, sn2 = acc
        b = bt // (N // T5)
        t = bt % (N // T5)
        base = b * 3 * N + t * T5
        hx = h_ref[:, pl.ds(base, T5)]
        hy = h_ref[:, pl.ds(base + N, T5)]
        hz = h_ref[:, pl.ds(base + 2 * N, T5)]
        h5x, h5y, h5z = _dot(w5, hx), _dot(w5, hy), _dot(w5, hz)
        nrm = jnp.sqrt(h5x * h5x + h5y * h5y + h5z * h5z) + EPS
        rin = 1.0 / nrm
        upd1 = jnp.stack([jnp.sum(h5x, axis=1), jnp.sum(h5y, axis=1),
                          jnp.sum(h5z, axis=1)], axis=1)
        upd2 = jnp.stack([jnp.sum(h5x * rin, axis=1), jnp.sum(h5y * rin, axis=1),
                          jnp.sum(h5z * rin, axis=1)], axis=1)
        bsel = (jax.lax.broadcasted_iota(jnp.int32, (1024, B * 3), 1) // 3
                == b).astype(jnp.float32)
        pad = jnp.zeros((1024, B * 3), jnp.float32)
        upd1 = pad + jnp.tile(upd1, (1, B)) * bsel
        upd2 = pad + jnp.tile(upd2, (1, B)) * bsel
        return (s1 + upd1, s2 + upd2,
                sn + jnp.sum(nrm, axis=1), sn2 + jnp.sum(nrm * nrm, axis=1))

    z13 = jnp.zeros((1024, B * 3), jnp.float32)
    z1 = jnp.zeros((1024,), jnp.float32)
    s1, s2, sn, sn2 = jax.lax.fori_loop(0, B * (N // T5), chunk,
                                        (z13, z13, z1, z1))
    cnt = float(B * N)
    mu = sn / cnt
    var = sn2 / cnt - mu * mu
    scale = jax.lax.rsqrt(var + 1e-5) / float(N)
    res = (s1 - mu[:, None] * s2) * scale[:, None]
    out_ref[...] = jnp.transpose(res.reshape(1024, B, 3), (1, 0, 2))


TK1 = 256
NSURV = 6


def _k1_body(xt_ref, xtile_ref, out_ref):
    IMIN = jnp.int32(-2147483648)
    xb = xt_ref[0]
    xtile = xtile_ref[0]
    pdt = _dot(jnp.transpose(xtile), xb) * 2.0
    xx_all = jnp.sum(xb * xb, axis=0, keepdims=True)
    xx_t = jnp.sum(xtile * xtile, axis=0, keepdims=True)
    pd = pdt - jnp.transpose(xx_t) - xx_all
    bits = jax.lax.bitcast_convert_type(pd, jnp.int32)
    key = jnp.where(bits >= 0, bits, bits ^ jnp.int32(0x7FFFFFFF))
    chunks = [key[:, c * 128:(c + 1) * 128] for c in range(N // 128)]
    lane = jax.lax.broadcasted_iota(jnp.int32, (TK1, 128), 1)
    ridx = [(N - 1) - c * 128 - lane for c in range(N // 128)]

    svs, svi = [], []
    for s in range(NSURV):
        m = functools.reduce(jnp.maximum, chunks)
        mi = functools.reduce(
            jnp.maximum,
            [jnp.where(v == m, r, IMIN) for v, r in zip(chunks, ridx)])
        svs.append(m)
        svi.append(mi)
        if s < NSURV - 1:
            chunks = [jnp.where(r == mi, IMIN, v)
                      for v, r in zip(chunks, ridx)]

    idxs = []
    for r in range(K):
        m6 = functools.reduce(jnp.maximum, svs)
        w = jnp.max(m6, axis=1, keepdims=True)
        wi = jnp.max(
            functools.reduce(
                jnp.maximum,
                [jnp.where(v == w, i, IMIN) for v, i in zip(svs, svi)]),
            axis=1, keepdims=True)
        idxs.append((N - 1) - wi + pl.program_id(0) * N)
        if r < K - 1:
            svs = [jnp.where(i == wi, IMIN, v) for v, i in zip(svs, svi)]
    out_ref[0] = jnp.concatenate(idxs, axis=1)


def _knn_idx(xt):
    return pl.pallas_call(
        _k1_body,
        grid=(B, N // TK1),
        in_specs=[
            pl.BlockSpec((1, 3, N), lambda b, j: (b, 0, 0)),
            pl.BlockSpec((1, 3, TK1), lambda b, j: (b, 0, j)),
        ],
        out_specs=pl.BlockSpec((1, TK1, K), lambda b, j: (b, j, 0)),
        out_shape=jax.ShapeDtypeStruct((B, N, K), jnp.int32),
    )(xt, xt)


def _sc_gather(tbl8, idxf):
    R = idxf.shape[0]
    NC, NS, L = 2, 16, 16
    bpw = R // (NC * NS)
    mesh = plsc.VectorSubcoreMesh(core_axis_name="c", subcore_axis_name="s")

    @functools.partial(
        pl.kernel, mesh=mesh,
        compiler_params=pltpu.CompilerParams(needs_layout_passes=False),
        out_type=jax.ShapeDtypeStruct((3 * R,), jnp.float32),
        scratch_types=[
            pltpu.VMEM((tbl8.shape[0],), jnp.float32),
            pltpu.VMEM((bpw,), jnp.int32),
            pltpu.VMEM((3 * bpw,), jnp.float32),
        ],
    )
    def k(tbl_hbm, idx_hbm, out_hbm, tbl_v, idx_v, outp_v):
        wid = lax.axis_index("s") * NC + lax.axis_index("c")
        base = wid * bpw
        pltpu.sync_copy(tbl_hbm, tbl_v)
        pltpu.sync_copy(idx_hbm.at[pl.ds(base, bpw)], idx_v)

        def chunk(i):
            ii = idx_v[pl.ds(i * L, L)] * 8
            outp_v[pl.ds(i * L, L)] = plsc.load_gather(tbl_v, [ii])
            outp_v[pl.ds(bpw + i * L, L)] = plsc.load_gather(tbl_v, [ii + 1])
            outp_v[pl.ds(2 * bpw + i * L, L)] = plsc.load_gather(tbl_v, [ii + 2])

        pl.loop(0, bpw // L)(chunk)
        for c in range(3):
            pltpu.sync_copy(outp_v.at[pl.ds(c * bpw, bpw)],
                            out_hbm.at[pl.ds(c * R + base, bpw)])

    return k(tbl8, idxf)


def kernel(x, W1f, W1d, W2f, W2d, W3f, W3d, W4f, W4d, W5):
    f32 = jnp.float32
    xt = jnp.transpose(x, (0, 2, 1))

    idx = _knn_idx(xt)

    tbl8 = jnp.pad(x.reshape(B * N, 3), ((0, 0), (0, 5))).reshape(-1)
    idx2 = jnp.transpose(idx, (0, 2, 1)).reshape(B * N * K)
    nbrT = _sc_gather(tbl8, idx2).reshape(3, B, K, N)

    stats = pl.pallas_call(
        _k3a_body,
        out_shape=jax.ShapeDtypeStruct((2, 64), f32),
    )(xt, nbrT, W1f)
    h1 = pl.pallas_call(
        _k3b_body,
        out_shape=jax.ShapeDtypeStruct((64, B, 3, N), f32),
    )(xt, nbrT, W1f, W1d, stats)

    out = pl.pallas_call(
        _k4_body,
        out_shape=jax.ShapeDtypeStruct((B, 1024, 3), f32),
        scratch_shapes=[
            pltpu.VMEM((128, M4), f32),
            pltpu.VMEM((128, M4), f32),
            pltpu.VMEM((128, M4), f32),
        ],
    )(h1.reshape(64, M4), W2f, W2d, W3f, W3d, W4f, W4d, W5)
    return out
